# baseline (speedup 1.0000x reference)
import functools

import jax
import jax.numpy as jnp
from jax import lax
from jax.experimental import pallas as pl
from jax.experimental.pallas import tpu as pltpu

_BF16 = jnp.bfloat16


def _tile_mask_build(w_ref, base, cin0, cin, ch, W, wc, dst, row_off, gates):
    wcin = W * cin
    pt = (lax.broadcasted_iota(jnp.int32, (wcin, cin), 0) // W ==
          lax.broadcasted_iota(jnp.int32, (wcin, cin), 1)).astype(_BF16)
    p = (lax.broadcasted_iota(jnp.int32, (ch, wc), 1) // W ==
         lax.broadcasted_iota(jnp.int32, (ch, wc), 0)).astype(_BF16)
    rb = lax.broadcasted_iota(jnp.int32, (wcin, wc), 0) % W
    cb = lax.broadcasted_iota(jnp.int32, (wcin, wc), 1) % W
    diag = rb - cb + 1
    for gi, g in enumerate(gates):
        for dy in range(3):
            acc = jnp.zeros((wcin, wc), _BF16)
            for dx in range(3):
                w16 = w_ref[(base + g) * 9 + dy * 3 + dx][
                    cin0:cin0 + cin, :].astype(_BF16)
                tiled = jnp.dot(
                    pt, jnp.dot(w16, p, preferred_element_type=jnp.float32
                                ).astype(_BF16),
                    preferred_element_type=jnp.float32).astype(_BF16)
                acc = acc + jnp.where(diag == dx, tiled, jnp.zeros_like(tiled))
            dst[row_off + dy * wcin:row_off + (dy + 1) * wcin,
                gi * wc:(gi + 1) * wc] = acc


def _gru_kernel(x_ref, h0_ref, w_ref, bias_ref, out_ref,
                h_ref, l_ref, wru_ref, wo_ref, *, H, W, Cx, Ch, WC):
    t = pl.program_id(1)
    BB = out_ref.shape[0]
    M = BB * H
    WCx = x_ref.shape[-1]
    KX = 3 * WCx

    @pl.when(t == 0)
    def _init():
        _tile_mask_build(w_ref, 0, 0, Cx, Ch, W, WC, wru_ref, 0, (0, 1))
        _tile_mask_build(w_ref, 0, Cx, Ch, Ch, W, WC, wru_ref, KX, (0, 1))
        _tile_mask_build(w_ref, 2, 0, Cx, Ch, W, WC, wo_ref, 0, (0,))
        _tile_mask_build(w_ref, 2, Cx, Ch, Ch, W, WC, wo_ref, KX, (0,))
        l_ref[...] = jnp.zeros(l_ref.shape, _BF16)
        h_ref[...] = h0_ref[...]
        h0b = h0_ref[...].astype(_BF16)
        l_ref[:, :, KX + WC:KX + 2 * WC] = h0b
        l_ref[:, 1:H, KX:KX + WC] = h0b[:, 0:H - 1, :]
        l_ref[:, 0:H - 1, KX + 2 * WC:] = h0b[:, 1:H, :]

    xv = x_ref[...].astype(_BF16)
    l_ref[:, :, WCx:2 * WCx] = xv
    l_ref[:, 1:H, 0:WCx] = xv[:, 0:H - 1, :]
    l_ref[:, 0:H - 1, 2 * WCx:KX] = xv[:, 1:H, :]

    hv = h_ref[...]
    acc_ru = (jnp.dot(l_ref[...].reshape(M, KX + 3 * WC), wru_ref[...],
                      preferred_element_type=jnp.float32)
              + bias_ref[:, :2 * WC])
    read_gate = jax.nn.sigmoid(acc_ru[:, :WC]).reshape(BB, H, WC)
    update_gate = jax.nn.sigmoid(acc_ru[:, WC:]).reshape(BB, H, WC)

    gated = (read_gate * hv).astype(_BF16)
    l_ref[:, :, KX + WC:KX + 2 * WC] = gated
    l_ref[:, 1:H, KX:KX + WC] = gated[:, 0:H - 1, :]
    l_ref[:, 0:H - 1, KX + 2 * WC:] = gated[:, 1:H, :]
    c = jnp.maximum(jnp.dot(l_ref[...].reshape(M, KX + 3 * WC), wo_ref[...],
                            preferred_element_type=jnp.float32)
                    + bias_ref[:, 2 * WC:], 0.0).reshape(BB, H, WC)

    new_h = update_gate * hv + (1.0 - update_gate) * c
    h_ref[...] = new_h
    out_ref[...] = new_h
    nb = new_h.astype(_BF16)
    l_ref[:, :, KX + WC:KX + 2 * WC] = nb
    l_ref[:, 1:H, KX:KX + WC] = nb[:, 0:H - 1, :]
    l_ref[:, 0:H - 1, KX + 2 * WC:] = nb[:, 1:H, :]


def kernel(x, h0, wr, br, wu, bu, wo, bo):
    T, B, H, W, Cx = x.shape
    Ch = h0.shape[-1]
    WCx, WC = W * Cx, W * Ch
    NC = 2
    BB = B // NC
    K = 3 * WCx + 3 * WC

    wcat = jnp.stack([wr, wu, wo]).reshape(27, Cx + Ch, Ch)
    bias = jnp.concatenate(
        [jnp.repeat(b, W) for b in (br, bu, bo)]).reshape(1, 3 * WC)

    xr = jnp.transpose(x, (0, 1, 2, 4, 3)).reshape(T, NC, BB, H, WCx)
    h0r = jnp.transpose(h0, (0, 1, 3, 2)).reshape(NC, BB, H, WC)

    out = pl.pallas_call(
        functools.partial(_gru_kernel, H=H, W=W, Cx=Cx, Ch=Ch, WC=WC),
        out_shape=jax.ShapeDtypeStruct((T, NC, BB, H, WC), x.dtype),
        grid=(NC, T),
        in_specs=[
            pl.BlockSpec((None, None, BB, H, WCx),
                         lambda c, t: (t, c, 0, 0, 0)),
            pl.BlockSpec((None, BB, H, WC), lambda c, t: (c, 0, 0, 0)),
            pl.BlockSpec((27, Cx + Ch, Ch), lambda c, t: (0, 0, 0)),
            pl.BlockSpec((1, 3 * WC), lambda c, t: (0, 0)),
        ],
        out_specs=pl.BlockSpec((None, None, BB, H, WC),
                               lambda c, t: (t, c, 0, 0, 0)),
        scratch_shapes=[
            pltpu.VMEM((BB, H, WC), jnp.float32),
            pltpu.VMEM((BB, H, K), _BF16),
            pltpu.VMEM((K, 2 * WC), _BF16),
            pltpu.VMEM((K, WC), _BF16),
        ],
        compiler_params=pltpu.CompilerParams(
            dimension_semantics=("parallel", "arbitrary"),
            vmem_limit_bytes=100 * 1024 * 1024,
        ),
    )(xr, h0r, wcat, bias)

    return jnp.transpose(out.reshape(T, B, H, Ch, W), (0, 1, 2, 4, 3))


# speedup vs baseline: 1.0187x; 1.0187x over previous
import functools

import jax
import jax.numpy as jnp
from jax import lax
from jax.experimental import pallas as pl
from jax.experimental.pallas import tpu as pltpu

_BF16 = jnp.bfloat16


def _tile_mask_build(w_ref, base, cin0, cin, ch, W, wc, dst, row_off, gates):
    wcin = W * cin
    pt = (lax.broadcasted_iota(jnp.int32, (wcin, cin), 0) // W ==
          lax.broadcasted_iota(jnp.int32, (wcin, cin), 1)).astype(_BF16)
    p = (lax.broadcasted_iota(jnp.int32, (ch, wc), 1) // W ==
         lax.broadcasted_iota(jnp.int32, (ch, wc), 0)).astype(_BF16)
    rb = lax.broadcasted_iota(jnp.int32, (wcin, wc), 0) % W
    cb = lax.broadcasted_iota(jnp.int32, (wcin, wc), 1) % W
    diag = rb - cb + 1
    for gi, g in enumerate(gates):
        for dy in range(3):
            acc = jnp.zeros((wcin, wc), _BF16)
            for dx in range(3):
                w16 = w_ref[(base + g) * 9 + dy * 3 + dx][
                    cin0:cin0 + cin, :].astype(_BF16)
                tiled = jnp.dot(
                    pt, jnp.dot(w16, p, preferred_element_type=jnp.float32
                                ).astype(_BF16),
                    preferred_element_type=jnp.float32).astype(_BF16)
                acc = acc + jnp.where(diag == dx, tiled, jnp.zeros_like(tiled))
            dst[row_off + dy * wcin:row_off + (dy + 1) * wcin,
                gi * wc:(gi + 1) * wc] = acc


def _gru_kernel(x_ref, h0_ref, w_ref, bias_ref, out_ref,
                h_ref, l_ref, wru_ref, wo_ref, *, H, W, Cx, Ch, WC):
    t = pl.program_id(1)
    BB = out_ref.shape[0]
    M = BB * H
    WCx = x_ref.shape[-1]
    KX = 3 * WCx

    @pl.when(t == 0)
    def _init():
        _tile_mask_build(w_ref, 0, 0, Cx, Ch, W, WC, wru_ref, 0, (0, 1))
        _tile_mask_build(w_ref, 0, Cx, Ch, Ch, W, WC, wru_ref, KX, (0, 1))
        _tile_mask_build(w_ref, 2, 0, Cx, Ch, W, WC, wo_ref, 0, (0,))
        _tile_mask_build(w_ref, 2, Cx, Ch, Ch, W, WC, wo_ref, KX, (0,))
        l_ref[...] = jnp.zeros(l_ref.shape, _BF16)
        h_ref[...] = h0_ref[...]

    xv = x_ref[...].astype(_BF16)
    l_ref[:, :, WCx:2 * WCx] = xv
    l_ref[:, 1:H, 0:WCx] = xv[:, 0:H - 1, :]
    l_ref[:, 0:H - 1, 2 * WCx:KX] = xv[:, 1:H, :]

    hv = h_ref[...]
    hb = hv.astype(_BF16)
    l_ref[:, :, KX + WC:KX + 2 * WC] = hb
    l_ref[:, 1:H, KX:KX + WC] = hb[:, 0:H - 1, :]
    l_ref[:, 0:H - 1, KX + 2 * WC:] = hb[:, 1:H, :]
    acc_ru = (jnp.dot(l_ref[...].reshape(M, KX + 3 * WC), wru_ref[...],
                      preferred_element_type=jnp.float32)
              + bias_ref[:, :2 * WC])
    read_gate = jax.nn.sigmoid(acc_ru[:, :WC]).reshape(BB, H, WC)
    update_gate = jax.nn.sigmoid(acc_ru[:, WC:]).reshape(BB, H, WC)

    gated = (read_gate * hv).astype(_BF16)
    l_ref[:, :, KX + WC:KX + 2 * WC] = gated
    l_ref[:, 1:H, KX:KX + WC] = gated[:, 0:H - 1, :]
    l_ref[:, 0:H - 1, KX + 2 * WC:] = gated[:, 1:H, :]
    c = jnp.maximum(jnp.dot(l_ref[...].reshape(M, KX + 3 * WC), wo_ref[...],
                            preferred_element_type=jnp.float32)
                    + bias_ref[:, 2 * WC:], 0.0).reshape(BB, H, WC)

    new_h = update_gate * hv + (1.0 - update_gate) * c
    h_ref[...] = new_h
    out_ref[...] = new_h


def kernel(x, h0, wr, br, wu, bu, wo, bo):
    T, B, H, W, Cx = x.shape
    Ch = h0.shape[-1]
    WCx, WC = W * Cx, W * Ch
    NC = 2
    BB = B // NC
    K = 3 * WCx + 3 * WC

    wcat = jnp.stack([wr, wu, wo]).reshape(27, Cx + Ch, Ch)
    bias = jnp.concatenate(
        [jnp.repeat(b, W) for b in (br, bu, bo)]).reshape(1, 3 * WC)

    xr = jnp.transpose(x, (0, 1, 2, 4, 3)).reshape(T, NC, BB, H, WCx)
    h0r = jnp.transpose(h0, (0, 1, 3, 2)).reshape(NC, BB, H, WC)

    out = pl.pallas_call(
        functools.partial(_gru_kernel, H=H, W=W, Cx=Cx, Ch=Ch, WC=WC),
        out_shape=jax.ShapeDtypeStruct((T, NC, BB, H, WC), x.dtype),
        grid=(NC, T),
        in_specs=[
            pl.BlockSpec((None, None, BB, H, WCx),
                         lambda c, t: (t, c, 0, 0, 0)),
            pl.BlockSpec((None, BB, H, WC), lambda c, t: (c, 0, 0, 0)),
            pl.BlockSpec((27, Cx + Ch, Ch), lambda c, t: (0, 0, 0)),
            pl.BlockSpec((1, 3 * WC), lambda c, t: (0, 0)),
        ],
        out_specs=pl.BlockSpec((None, None, BB, H, WC),
                               lambda c, t: (t, c, 0, 0, 0)),
        scratch_shapes=[
            pltpu.VMEM((BB, H, WC), jnp.float32),
            pltpu.VMEM((BB, H, K), _BF16),
            pltpu.VMEM((K, 2 * WC), _BF16),
            pltpu.VMEM((K, WC), _BF16),
        ],
        compiler_params=pltpu.CompilerParams(
            dimension_semantics=("parallel", "arbitrary"),
            vmem_limit_bytes=100 * 1024 * 1024,
        ),
    )(xr, h0r, wcat, bias)

    return jnp.transpose(out.reshape(T, B, H, Ch, W), (0, 1, 2, 4, 3))
